# baseline (speedup 1.0000x reference)
import functools
import math

import jax
import jax.numpy as jnp
from jax import lax
from jax.experimental import pallas as pl
from jax.experimental.pallas import tpu as pltpu
from jax.experimental.pallas import tpu_sc as plsc

N = 10000
NP = 10240
H = 128
E = 320000
G = 1000
GP = 1024

NC = 2
NS = 16
EPT = E // (NC * NS)
C = 80
NCHUNK = EPT // C
RPT = NP // NS
ZR = 64

@functools.lru_cache(maxsize=None)
def _sc_kernels():
    mesh = plsc.VectorSubcoreMesh(core_axis_name="c", subcore_axis_name="s",
                                  num_cores=NC, num_subcores=NS)
    deg = functools.partial(
        pl.kernel,
        out_type=jax.ShapeDtypeStruct((NC, NP, H), jnp.float32),
        mesh=mesh,
        scratch_types=[
            pltpu.VMEM((C, H), jnp.float32),
            pltpu.VMEM((ZR, H), jnp.float32),
            pltpu.VMEM((NCHUNK, C), jnp.int32),
            pltpu.VMEM_SHARED((NP, H), jnp.float32),
            pltpu.SemaphoreType.DMA,
        ],
    )(_sc_degree_body)
    scratch = [
        pltpu.VMEM((5, C), jnp.int32),
        pltpu.VMEM((5, C), jnp.int32),
        pltpu.VMEM((3, C, H), jnp.float32),
        pltpu.VMEM((GP // NS,), jnp.int32),
        pltpu.VMEM((GP // NS, H), jnp.float32),
        pltpu.VMEM((GP // (NC * NS),), jnp.int32),
        pltpu.VMEM((GP // (NC * NS), H), jnp.float32),
        pltpu.VMEM_SHARED((NP, H), jnp.float32),
        pltpu.SemaphoreType.DMA,
        pltpu.SemaphoreType.DMA,
        pltpu.SemaphoreType.DMA,
    ]
    layer = functools.partial(
        pl.kernel,
        out_type=jax.ShapeDtypeStruct((NC, NP, H), jnp.float32),
        mesh=mesh,
        scratch_types=scratch,
    )(functools.partial(_sc_layer_body, False))
    layer3 = functools.partial(
        pl.kernel,
        out_type=(
            jax.ShapeDtypeStruct((NC, GP, H), jnp.float32),
            jax.ShapeDtypeStruct((GP, H), jnp.float32),
            jax.ShapeDtypeStruct((GP, H), jnp.float32),
            jax.ShapeDtypeStruct((GP, H), jnp.float32),
        ),
        mesh=mesh,
        scratch_types=scratch,
    )(functools.partial(_sc_layer_body, True))
    return deg, layer, layer3


def _sc_degree_body(dst_hbm, ones_hbm, zeros_hbm, out_hbm, ones_v, stage_v,
                    didx, deg_sh, sem):
    c = lax.axis_index("c")
    s = lax.axis_index("s")
    wid = c * NS + s
    pltpu.sync_copy(dst_hbm.at[wid], didx)
    pltpu.sync_copy(zeros_hbm, stage_v)
    for k in range(RPT // ZR):
        pltpu.sync_copy(stage_v, deg_sh.at[pl.ds(s * RPT + k * ZR, ZR)])
    pltpu.sync_copy(ones_hbm, ones_v)
    plsc.subcore_barrier()

    def body(t, carry):
        pltpu.async_copy(ones_v, deg_sh.at[didx.at[t]], sem, add=True)

        @pl.when(t >= 1)
        def _():
            pltpu.make_async_copy(ones_v, deg_sh.at[didx.at[t - 1]],
                                  sem).wait()

        return carry

    lax.fori_loop(0, NCHUNK, body, 0)
    pltpu.make_async_copy(ones_v, deg_sh.at[didx.at[NCHUNK - 1]], sem).wait()
    plsc.subcore_barrier()
    for k in range(RPT // ZR):
        pltpu.sync_copy(deg_sh.at[pl.ds(s * RPT + k * ZR, ZR)], stage_v)
        pltpu.sync_copy(stage_v, out_hbm.at[c, pl.ds(s * RPT + k * ZR, ZR)])


def _sc_layer_body(last, g_hbm, src_hbm, dst_hbm, zeros_hbm, gi_hbm,
                   hprev_hbm, dinv_hbm, *rest):
    if last:
        (aggg_hbm, hg_hbm, gg_hbm, dvg_hbm,
         sidx, didx, rows, gidx, gbuf, gidx2, hbuf,
         agg_sh, isem, gsem, ssem) = rest
    else:
        (agg_hbm,
         sidx, didx, rows, gidx, gbuf, gidx2, hbuf,
         agg_sh, isem, gsem, ssem) = rest
    c = lax.axis_index("c")
    s = lax.axis_index("s")
    wid = c * NS + s

    pltpu.sync_copy(zeros_hbm, rows.at[0])
    for k in range(RPT // C):
        pltpu.async_copy(rows.at[0], agg_sh.at[pl.ds(s * RPT + k * C, C)],
                         gsem)
    for k in range(RPT // C):
        pltpu.make_async_copy(rows.at[0],
                              agg_sh.at[pl.ds(s * RPT + k * C, C)],
                              gsem).wait()
    plsc.subcore_barrier()

    def idx_load(t, slot):
        pltpu.async_copy(src_hbm.at[wid, t], sidx.at[slot], isem)
        pltpu.async_copy(dst_hbm.at[wid, t], didx.at[slot], isem)

    def idx_wait(t, slot):
        pltpu.make_async_copy(src_hbm.at[wid, t], sidx.at[slot], isem).wait()
        pltpu.make_async_copy(dst_hbm.at[wid, t], didx.at[slot], isem).wait()

    def chunk(t, head=False, tail=False):
        r = lax.rem(t, 3)
        r2 = lax.rem(t + 2, 3)
        s5 = lax.rem(t, 5)
        pltpu.make_async_copy(g_hbm.at[sidx.at[s5]], rows.at[r], gsem).wait()
        pltpu.async_copy(rows.at[r], agg_sh.at[didx.at[s5]], ssem, add=True)
        if head:
            @pl.when(t >= 1)
            def _():
                pltpu.make_async_copy(rows.at[r2],
                                      agg_sh.at[didx.at[lax.rem(t + 4, 5)]],
                                      ssem).wait()
        else:
            pltpu.make_async_copy(rows.at[r2],
                                  agg_sh.at[didx.at[lax.rem(t + 4, 5)]],
                                  ssem).wait()
        if tail:
            @pl.when(t < NCHUNK - 2)
            def _():
                idx_wait(t, lax.rem(t + 2, 5))
                pltpu.async_copy(g_hbm.at[sidx.at[lax.rem(t + 2, 5)]],
                                 rows.at[r2], gsem)

            @pl.when(t < NCHUNK - 4)
            def _():
                idx_load(t + 4, lax.rem(t + 4, 5))
        else:
            idx_wait(t, lax.rem(t + 2, 5))
            pltpu.async_copy(g_hbm.at[sidx.at[lax.rem(t + 2, 5)]],
                             rows.at[r2], gsem)
            idx_load(t + 4, lax.rem(t + 4, 5))

    def fhead(t, carry):
        chunk(t, head=True)
        return carry

    def fmid(i, carry):
        t0 = i * 5
        for j in range(5):
            chunk(t0 + j)
        return carry

    def ftail(t, carry):
        chunk(t, tail=True)
        return carry

    def iload(t, carry):
        idx_load(t, t)
        return carry

    lax.fori_loop(0, 4, iload, 0)
    idx_wait(jnp.int32(0), jnp.int32(0))
    idx_wait(jnp.int32(1), jnp.int32(1))
    z = jnp.int32(0)
    o = jnp.int32(1)
    pltpu.async_copy(g_hbm.at[sidx.at[z]], rows.at[z], gsem)
    pltpu.async_copy(g_hbm.at[sidx.at[o]], rows.at[o], gsem)

    lax.fori_loop(0, 5, fhead, 0)
    lax.fori_loop(1, (NCHUNK - 5) // 5, fmid, 0)
    lax.fori_loop(NCHUNK - 5, NCHUNK, ftail, 0)

    last_t = jnp.int32(NCHUNK - 1)
    pltpu.make_async_copy(rows.at[lax.rem(last_t, 3)],
                          agg_sh.at[didx.at[lax.rem(last_t, 5)]],
                          ssem).wait()
    plsc.subcore_barrier()

    if not last:
        for k in range(RPT // C):
            if k >= 3:
                pltpu.make_async_copy(
                    rows.at[(k - 3) % 3],
                    agg_hbm.at[c, pl.ds(s * RPT + (k - 3) * C, C)],
                    ssem).wait()
            pltpu.async_copy(agg_sh.at[pl.ds(s * RPT + k * C, C)],
                             rows.at[k % 3], gsem).wait()
            pltpu.async_copy(rows.at[k % 3],
                             agg_hbm.at[c, pl.ds(s * RPT + k * C, C)], ssem)
        for k in range(RPT // C - 3, RPT // C):
            pltpu.make_async_copy(
                rows.at[k % 3], agg_hbm.at[c, pl.ds(s * RPT + k * C, C)],
                ssem).wait()
        return

    gpc = GP // NS
    pltpu.sync_copy(gi_hbm.at[pl.ds(s * gpc, gpc)], gidx)
    pltpu.async_copy(agg_sh.at[gidx], gbuf, gsem).wait()
    pltpu.sync_copy(gbuf, aggg_hbm.at[c, pl.ds(s * gpc, gpc)])

    gpw = GP // (NC * NS)
    pltpu.sync_copy(gi_hbm.at[pl.ds(wid * gpw, gpw)], gidx2)
    pltpu.async_copy(hprev_hbm.at[gidx2], hbuf, gsem).wait()
    pltpu.sync_copy(hbuf, hg_hbm.at[pl.ds(wid * gpw, gpw)])
    pltpu.async_copy(g_hbm.at[gidx2], hbuf, gsem).wait()
    pltpu.sync_copy(hbuf, gg_hbm.at[pl.ds(wid * gpw, gpw)])
    pltpu.async_copy(dinv_hbm.at[gidx2], hbuf, gsem).wait()
    pltpu.sync_copy(hbuf, dvg_hbm.at[pl.ds(wid * gpw, gpw)])


_R = 2048


def _tc_a_body(x_ref, w0_ref, b0_ref, w1_ref, degp_ref,
               h0_ref, g1_ref, dinv_ref):
    h0 = jnp.maximum(
        jnp.dot(x_ref[...], w0_ref[...], preferred_element_type=jnp.float32)
        + b0_ref[...], 0.0)
    deg = degp_ref[0, :, 0:1] + degp_ref[1, :, 0:1] + 1.0
    dinv = lax.rsqrt(deg)
    h0_ref[...] = h0
    g1_ref[...] = jnp.dot(h0, w1_ref[...],
                          preferred_element_type=jnp.float32) * dinv
    dinv_ref[...] = jnp.broadcast_to(dinv, (dinv.shape[0], H))


def _tc_a(x, w0, b0, w1, degp):
    grid = (NP // _R,)
    return pl.pallas_call(
        _tc_a_body,
        grid=grid,
        in_specs=[
            pl.BlockSpec((_R, H), lambda i: (i, 0)),
            pl.BlockSpec((H, H), lambda i: (0, 0)),
            pl.BlockSpec((1, H), lambda i: (0, 0)),
            pl.BlockSpec((H, H), lambda i: (0, 0)),
            pl.BlockSpec((NC, _R, H), lambda i: (0, i, 0)),
        ],
        out_specs=[
            pl.BlockSpec((_R, H), lambda i: (i, 0)),
            pl.BlockSpec((_R, H), lambda i: (i, 0)),
            pl.BlockSpec((_R, H), lambda i: (i, 0)),
        ],
        out_shape=[
            jax.ShapeDtypeStruct((NP, H), jnp.float32),
            jax.ShapeDtypeStruct((NP, H), jnp.float32),
            jax.ShapeDtypeStruct((NP, H), jnp.float32),
        ],
    )(x, w0, b0, w1, degp)


def _tc_b_body(hprev_ref, g_ref, parts_ref, dinv_ref, b_ref, w_ref,
               hnew_ref, gnext_ref):
    dinv = dinv_ref[:, 0:1]
    agg = dinv * (parts_ref[0] + parts_ref[1] + g_ref[...]) + b_ref[...]
    hn = hprev_ref[...] + jnp.maximum(agg, 0.0)
    hnew_ref[...] = hn
    gnext_ref[...] = jnp.dot(hn, w_ref[...],
                             preferred_element_type=jnp.float32) * dinv


def _tc_b(hprev, g, parts, dinv, b, w):
    grid = (NP // _R,)
    return pl.pallas_call(
        _tc_b_body,
        grid=grid,
        in_specs=[
            pl.BlockSpec((_R, H), lambda i: (i, 0)),
            pl.BlockSpec((_R, H), lambda i: (i, 0)),
            pl.BlockSpec((NC, _R, H), lambda i: (0, i, 0)),
            pl.BlockSpec((_R, H), lambda i: (i, 0)),
            pl.BlockSpec((1, H), lambda i: (0, 0)),
            pl.BlockSpec((H, H), lambda i: (0, 0)),
        ],
        out_specs=[
            pl.BlockSpec((_R, H), lambda i: (i, 0)),
            pl.BlockSpec((_R, H), lambda i: (i, 0)),
        ],
        out_shape=[
            jax.ShapeDtypeStruct((NP, H), jnp.float32),
            jax.ShapeDtypeStruct((NP, H), jnp.float32),
        ],
    )(hprev, g, parts, dinv, b, w)


def _tc_heads_body(aggg_ref, hg_ref, gg_ref, dvg_ref, b3_ref,
                   w1_ref, b1_ref, w2_ref, b2_ref, w3_ref, b3c_ref, out_ref):
    dinv = dvg_ref[:, 0:1]
    agg = dinv * (aggg_ref[0] + aggg_ref[1] + gg_ref[...]) + b3_ref[...]
    xg = hg_ref[...] + jnp.maximum(agg, 0.0)
    t = jnp.maximum(
        jnp.dot(xg, w1_ref[...], preferred_element_type=jnp.float32)
        + b1_ref[...], 0.0)
    t = jnp.maximum(
        jnp.dot(t, w2_ref[...], preferred_element_type=jnp.float32)
        + b2_ref[...], 0.0)
    out_ref[...] = (jnp.dot(t, w3_ref[...], preferred_element_type=jnp.float32)
                    + b3c_ref[...])


def _tc_heads(aggg, hg, gg, dvg, b3, w1, b1, w2, b2, w3, b3c):
    return pl.pallas_call(
        _tc_heads_body,
        out_shape=jax.ShapeDtypeStruct((GP, H), jnp.float32),
    )(aggg, hg, gg, dvg, b3, w1, b1, w2, b2, w3, b3c)


def _fold(params):
    s = 1.0 / math.sqrt(1.0 + 1e-5)

    def bnf(wb):
        return s * wb[0], wb[1]

    wi, bi = bnf(params['in_bn'])
    W, B = params['in_lin']
    w0s, b0s = bnf(params['bn0'])
    W0 = (wi[:, None] * W) * w0s[None, :]
    b0 = ((bi @ W) + B) * w0s + b0s

    convs = []
    for i in (1, 2, 3):
        Wc, Bc = params['conv%d' % i]
        ws, bs = bnf(params['bn%d' % i])
        convs.append((Wc * ws[None, :], Bc * ws + bs))

    hnames = ('home_goals', 'away_goals', 'home_shots', 'away_shots')
    W1 = jnp.zeros((H, 384), jnp.float32)
    b1 = jnp.zeros((384,), jnp.float32)
    W2 = jnp.zeros((384, 256), jnp.float32)
    b2 = jnp.zeros((256,), jnp.float32)
    W3 = jnp.zeros((256, H), jnp.float32)
    b3 = jnp.zeros((H,), jnp.float32)
    for i, nme in enumerate(hnames):
        p = params['heads'][nme]
        ws, bs = bnf(p['bn'])
        W1 = W1.at[:, 64 * i:64 * i + 64].set(p['l1'][0] * ws[None, :])
        b1 = b1.at[64 * i:64 * i + 64].set(p['l1'][1] * ws + bs)
        W2 = W2.at[64 * i:64 * i + 64, 32 * i:32 * i + 32].set(p['l2'][0])
        b2 = b2.at[32 * i:32 * i + 32].set(p['l2'][1])
        W3 = W3.at[32 * i:32 * i + 32, i:i + 1].set(p['l3'][0])
        b3 = b3.at[i].set(p['l3'][1][0])
    po = params['outcome']
    w1s, b1s = bnf(po['bn1'])
    w2s, b2s = bnf(po['bn2'])
    W1 = W1.at[:, 256:320].set(po['l1'][0] * w1s[None, :])
    b1 = b1.at[256:320].set(po['l1'][1] * w1s + b1s)
    W2 = W2.at[256:320, 128:160].set(po['l2'][0] * w2s[None, :])
    b2 = b2.at[128:160].set(po['l2'][1] * w2s + b2s)
    W3 = W3.at[128:160, 4:7].set(po['l3'][0])
    b3 = b3.at[4:7].set(po['l3'][1])

    return (W0, b0[None, :], convs,
            W1, b1[None, :], W2, b2[None, :], W3, b3[None, :])


def kernel(x, edge_index, game_indices, params):
    src = edge_index[0].reshape(NC * NS, NCHUNK, C)
    dst = edge_index[1].reshape(NC * NS, NCHUNK, C)
    x = jnp.pad(x, ((0, NP - N), (0, 0)))
    gi = jnp.pad(game_indices.astype(jnp.int32), (0, GP - G))

    (W0, b0, convs, W1c, b1c, W2c, b2c, W3c, b3c) = _fold(params)
    (W1f, b1f), (W2f, b2f), (W3f, b3f) = convs

    onesr = jnp.ones((C, H), jnp.float32)
    zrows = jnp.zeros((C, H), jnp.float32)
    zdeg = jnp.zeros((ZR, H), jnp.float32)

    _sc_degree, _sc_layer, _sc_layer3 = _sc_kernels()
    degp = _sc_degree(dst, onesr, zdeg)
    h0, g1, dinv = _tc_a(x, W0, b0, W1f, degp)

    parts1 = _sc_layer(g1, src, dst, zrows, gi, h0, dinv)
    h1, g2 = _tc_b(h0, g1, parts1, dinv, b1f[None, :], W2f)

    parts2 = _sc_layer(g2, src, dst, zrows, gi, h1, dinv)
    h2, g3 = _tc_b(h1, g2, parts2, dinv, b2f[None, :], W3f)

    aggg, hg, gg, dvg = _sc_layer3(g3, src, dst, zrows, gi, h2, dinv)
    out = _tc_heads(aggg, hg, gg, dvg, b3f[None, :],
                    W1c, b1c, W2c, b2c, W3c, b3c)

    o = out[:G]
    return (o[:, 0:1], o[:, 1:2], o[:, 2:3], o[:, 3:4], o[:, 4:7])


# speedup vs baseline: 1.0018x; 1.0018x over previous
import functools
import math

import jax
import jax.numpy as jnp
from jax import lax
from jax.experimental import pallas as pl
from jax.experimental.pallas import tpu as pltpu
from jax.experimental.pallas import tpu_sc as plsc

N = 10000
NP = 10240
H = 128
E = 320000
G = 1000
GP = 1024

NC = 2
NS = 16
EPT = E // (NC * NS)
C = 80
NCHUNK = EPT // C
RPT = NP // NS
ZR = 64

@functools.lru_cache(maxsize=None)
def _sc_kernels():
    mesh = plsc.VectorSubcoreMesh(core_axis_name="c", subcore_axis_name="s",
                                  num_cores=NC, num_subcores=NS)
    deg = functools.partial(
        pl.kernel,
        out_type=jax.ShapeDtypeStruct((NC, NP, H), jnp.float32),
        mesh=mesh,
        scratch_types=[
            pltpu.VMEM((C, H), jnp.float32),
            pltpu.VMEM((ZR, H), jnp.float32),
            pltpu.VMEM((NCHUNK, C), jnp.int32),
            pltpu.VMEM_SHARED((NP, H), jnp.float32),
            pltpu.SemaphoreType.DMA,
        ],
    )(_sc_degree_body)
    scratch = [
        pltpu.VMEM((5, C), jnp.int32),
        pltpu.VMEM((5, C), jnp.int32),
        pltpu.VMEM((3, C, H), jnp.float32),
        pltpu.VMEM((GP // NS,), jnp.int32),
        pltpu.VMEM((GP // NS, H), jnp.float32),
        pltpu.VMEM((GP // (NC * NS),), jnp.int32),
        pltpu.VMEM((GP // (NC * NS), H), jnp.float32),
        pltpu.VMEM_SHARED((NP, H), jnp.float32),
        pltpu.SemaphoreType.DMA,
        pltpu.SemaphoreType.DMA,
        pltpu.SemaphoreType.DMA,
    ]
    layer = functools.partial(
        pl.kernel,
        out_type=jax.ShapeDtypeStruct((NC, NP, H), jnp.float32),
        mesh=mesh,
        scratch_types=scratch,
    )(functools.partial(_sc_layer_body, False))
    layer3 = functools.partial(
        pl.kernel,
        out_type=(
            jax.ShapeDtypeStruct((NC, GP, H), jnp.float32),
            jax.ShapeDtypeStruct((GP, H), jnp.float32),
            jax.ShapeDtypeStruct((GP, H), jnp.float32),
            jax.ShapeDtypeStruct((GP, H), jnp.float32),
        ),
        mesh=mesh,
        scratch_types=scratch,
    )(functools.partial(_sc_layer_body, True))
    return deg, layer, layer3


def _sc_degree_body(dst_hbm, ones_hbm, zeros_hbm, out_hbm, ones_v, stage_v,
                    didx, deg_sh, sem):
    c = lax.axis_index("c")
    s = lax.axis_index("s")
    wid = c * NS + s
    pltpu.sync_copy(dst_hbm.at[wid], didx)
    pltpu.sync_copy(zeros_hbm, stage_v)
    for k in range(RPT // ZR):
        pltpu.sync_copy(stage_v, deg_sh.at[pl.ds(s * RPT + k * ZR, ZR)])
    pltpu.sync_copy(ones_hbm, ones_v)
    plsc.subcore_barrier()

    def body(t, carry):
        pltpu.async_copy(ones_v, deg_sh.at[didx.at[t]], sem, add=True)

        @pl.when(t >= 1)
        def _():
            pltpu.make_async_copy(ones_v, deg_sh.at[didx.at[t - 1]],
                                  sem).wait()

        return carry

    lax.fori_loop(0, NCHUNK, body, 0)
    pltpu.make_async_copy(ones_v, deg_sh.at[didx.at[NCHUNK - 1]], sem).wait()
    plsc.subcore_barrier()
    for k in range(RPT // ZR):
        pltpu.sync_copy(deg_sh.at[pl.ds(s * RPT + k * ZR, ZR)], stage_v)
        pltpu.sync_copy(stage_v, out_hbm.at[c, pl.ds(s * RPT + k * ZR, ZR)])


def _sc_layer_body(last, g_hbm, src_hbm, dst_hbm, zeros_hbm, gi_hbm,
                   hprev_hbm, dinv_hbm, *rest):
    if last:
        (aggg_hbm, hg_hbm, gg_hbm, dvg_hbm,
         sidx, didx, rows, gidx, gbuf, gidx2, hbuf,
         agg_sh, isem, gsem, ssem) = rest
    else:
        (agg_hbm,
         sidx, didx, rows, gidx, gbuf, gidx2, hbuf,
         agg_sh, isem, gsem, ssem) = rest
    c = lax.axis_index("c")
    s = lax.axis_index("s")
    wid = c * NS + s

    pltpu.sync_copy(zeros_hbm, rows.at[0])
    for k in range(RPT // C):
        pltpu.async_copy(rows.at[0], agg_sh.at[pl.ds(s * RPT + k * C, C)],
                         gsem)
    for k in range(RPT // C):
        pltpu.make_async_copy(rows.at[0],
                              agg_sh.at[pl.ds(s * RPT + k * C, C)],
                              gsem).wait()
    plsc.subcore_barrier()

    def idx_load(t, slot):
        pltpu.async_copy(src_hbm.at[wid, t], sidx.at[slot], isem)
        pltpu.async_copy(dst_hbm.at[wid, t], didx.at[slot], isem)

    def idx_wait(slot):
        pltpu.make_async_copy(src_hbm.at[wid, 0], sidx.at[slot], isem).wait()
        pltpu.make_async_copy(dst_hbm.at[wid, 0], didx.at[slot], isem).wait()

    for t in range(4):
        idx_load(t, t)
    idx_wait(0)
    idx_wait(1)
    pltpu.async_copy(g_hbm.at[sidx.at[0]], rows.at[0], gsem)
    pltpu.async_copy(g_hbm.at[sidx.at[1]], rows.at[1], gsem)

    def body(t, carry):
        r = lax.rem(t, 3)
        s5 = lax.rem(t, 5)
        pltpu.make_async_copy(g_hbm.at[sidx.at[s5]], rows.at[r], gsem).wait()
        pltpu.async_copy(rows.at[r], agg_sh.at[didx.at[s5]], ssem, add=True)

        @pl.when(t >= 1)
        def _():
            pltpu.make_async_copy(rows.at[lax.rem(t + 2, 3)],
                                  agg_sh.at[didx.at[lax.rem(t + 4, 5)]],
                                  ssem).wait()

        @pl.when(t < NCHUNK - 2)
        def _():
            idx_wait(lax.rem(t + 2, 5))
            pltpu.async_copy(g_hbm.at[sidx.at[lax.rem(t + 2, 5)]],
                             rows.at[lax.rem(t + 2, 3)], gsem)

        @pl.when(t < NCHUNK - 4)
        def _():
            idx_load(t + 4, lax.rem(t + 4, 5))

        return carry

    lax.fori_loop(0, NCHUNK, body, 0)
    pltpu.make_async_copy(rows.at[(NCHUNK - 1) % 3],
                          agg_sh.at[didx.at[(NCHUNK - 1) % 5]], ssem).wait()
    plsc.subcore_barrier()

    if not last:
        for k in range(RPT // C):
            if k >= 3:
                pltpu.make_async_copy(
                    rows.at[(k - 3) % 3],
                    agg_hbm.at[c, pl.ds(s * RPT + (k - 3) * C, C)],
                    ssem).wait()
            pltpu.async_copy(agg_sh.at[pl.ds(s * RPT + k * C, C)],
                             rows.at[k % 3], gsem).wait()
            pltpu.async_copy(rows.at[k % 3],
                             agg_hbm.at[c, pl.ds(s * RPT + k * C, C)], ssem)
        for k in range(RPT // C - 3, RPT // C):
            pltpu.make_async_copy(
                rows.at[k % 3], agg_hbm.at[c, pl.ds(s * RPT + k * C, C)],
                ssem).wait()
        return

    gpc = GP // NS
    pltpu.sync_copy(gi_hbm.at[pl.ds(s * gpc, gpc)], gidx)
    pltpu.async_copy(agg_sh.at[gidx], gbuf, gsem).wait()
    pltpu.sync_copy(gbuf, aggg_hbm.at[c, pl.ds(s * gpc, gpc)])

    gpw = GP // (NC * NS)
    pltpu.sync_copy(gi_hbm.at[pl.ds(wid * gpw, gpw)], gidx2)
    pltpu.async_copy(hprev_hbm.at[gidx2], hbuf, gsem).wait()
    pltpu.sync_copy(hbuf, hg_hbm.at[pl.ds(wid * gpw, gpw)])
    pltpu.async_copy(g_hbm.at[gidx2], hbuf, gsem).wait()
    pltpu.sync_copy(hbuf, gg_hbm.at[pl.ds(wid * gpw, gpw)])
    pltpu.async_copy(dinv_hbm.at[gidx2], hbuf, gsem).wait()
    pltpu.sync_copy(hbuf, dvg_hbm.at[pl.ds(wid * gpw, gpw)])


_R = 2048


def _tc_a_body(x_ref, w0_ref, b0_ref, w1_ref, degp_ref,
               h0_ref, g1_ref, dinv_ref):
    h0 = jnp.maximum(
        jnp.dot(x_ref[...], w0_ref[...], preferred_element_type=jnp.float32)
        + b0_ref[...], 0.0)
    deg = degp_ref[0, :, 0:1] + degp_ref[1, :, 0:1] + 1.0
    dinv = lax.rsqrt(deg)
    h0_ref[...] = h0
    g1_ref[...] = jnp.dot(h0, w1_ref[...],
                          preferred_element_type=jnp.float32) * dinv
    dinv_ref[...] = jnp.broadcast_to(dinv, (dinv.shape[0], H))


def _tc_a(x, w0, b0, w1, degp):
    grid = (NP // _R,)
    return pl.pallas_call(
        _tc_a_body,
        grid=grid,
        in_specs=[
            pl.BlockSpec((_R, H), lambda i: (i, 0)),
            pl.BlockSpec((H, H), lambda i: (0, 0)),
            pl.BlockSpec((1, H), lambda i: (0, 0)),
            pl.BlockSpec((H, H), lambda i: (0, 0)),
            pl.BlockSpec((NC, _R, H), lambda i: (0, i, 0)),
        ],
        out_specs=[
            pl.BlockSpec((_R, H), lambda i: (i, 0)),
            pl.BlockSpec((_R, H), lambda i: (i, 0)),
            pl.BlockSpec((_R, H), lambda i: (i, 0)),
        ],
        out_shape=[
            jax.ShapeDtypeStruct((NP, H), jnp.float32),
            jax.ShapeDtypeStruct((NP, H), jnp.float32),
            jax.ShapeDtypeStruct((NP, H), jnp.float32),
        ],
    )(x, w0, b0, w1, degp)


def _tc_b_body(hprev_ref, g_ref, parts_ref, dinv_ref, b_ref, w_ref,
               hnew_ref, gnext_ref):
    dinv = dinv_ref[:, 0:1]
    agg = dinv * (parts_ref[0] + parts_ref[1] + g_ref[...]) + b_ref[...]
    hn = hprev_ref[...] + jnp.maximum(agg, 0.0)
    hnew_ref[...] = hn
    gnext_ref[...] = jnp.dot(hn, w_ref[...],
                             preferred_element_type=jnp.float32) * dinv


def _tc_b(hprev, g, parts, dinv, b, w):
    grid = (NP // _R,)
    return pl.pallas_call(
        _tc_b_body,
        grid=grid,
        in_specs=[
            pl.BlockSpec((_R, H), lambda i: (i, 0)),
            pl.BlockSpec((_R, H), lambda i: (i, 0)),
            pl.BlockSpec((NC, _R, H), lambda i: (0, i, 0)),
            pl.BlockSpec((_R, H), lambda i: (i, 0)),
            pl.BlockSpec((1, H), lambda i: (0, 0)),
            pl.BlockSpec((H, H), lambda i: (0, 0)),
        ],
        out_specs=[
            pl.BlockSpec((_R, H), lambda i: (i, 0)),
            pl.BlockSpec((_R, H), lambda i: (i, 0)),
        ],
        out_shape=[
            jax.ShapeDtypeStruct((NP, H), jnp.float32),
            jax.ShapeDtypeStruct((NP, H), jnp.float32),
        ],
    )(hprev, g, parts, dinv, b, w)


def _tc_heads_body(aggg_ref, hg_ref, gg_ref, dvg_ref, b3_ref,
                   w1_ref, b1_ref, w2_ref, b2_ref, w3_ref, b3c_ref, out_ref):
    dinv = dvg_ref[:, 0:1]
    agg = dinv * (aggg_ref[0] + aggg_ref[1] + gg_ref[...]) + b3_ref[...]
    xg = hg_ref[...] + jnp.maximum(agg, 0.0)
    t = jnp.maximum(
        jnp.dot(xg, w1_ref[...], preferred_element_type=jnp.float32)
        + b1_ref[...], 0.0)
    t = jnp.maximum(
        jnp.dot(t, w2_ref[...], preferred_element_type=jnp.float32)
        + b2_ref[...], 0.0)
    out_ref[...] = (jnp.dot(t, w3_ref[...], preferred_element_type=jnp.float32)
                    + b3c_ref[...])


def _tc_heads(aggg, hg, gg, dvg, b3, w1, b1, w2, b2, w3, b3c):
    return pl.pallas_call(
        _tc_heads_body,
        out_shape=jax.ShapeDtypeStruct((GP, H), jnp.float32),
    )(aggg, hg, gg, dvg, b3, w1, b1, w2, b2, w3, b3c)


def _fold(params):
    s = 1.0 / math.sqrt(1.0 + 1e-5)

    def bnf(wb):
        return s * wb[0], wb[1]

    wi, bi = bnf(params['in_bn'])
    W, B = params['in_lin']
    w0s, b0s = bnf(params['bn0'])
    W0 = (wi[:, None] * W) * w0s[None, :]
    b0 = ((bi @ W) + B) * w0s + b0s

    convs = []
    for i in (1, 2, 3):
        Wc, Bc = params['conv%d' % i]
        ws, bs = bnf(params['bn%d' % i])
        convs.append((Wc * ws[None, :], Bc * ws + bs))

    hnames = ('home_goals', 'away_goals', 'home_shots', 'away_shots')
    W1 = jnp.zeros((H, 384), jnp.float32)
    b1 = jnp.zeros((384,), jnp.float32)
    W2 = jnp.zeros((384, 256), jnp.float32)
    b2 = jnp.zeros((256,), jnp.float32)
    W3 = jnp.zeros((256, H), jnp.float32)
    b3 = jnp.zeros((H,), jnp.float32)
    for i, nme in enumerate(hnames):
        p = params['heads'][nme]
        ws, bs = bnf(p['bn'])
        W1 = W1.at[:, 64 * i:64 * i + 64].set(p['l1'][0] * ws[None, :])
        b1 = b1.at[64 * i:64 * i + 64].set(p['l1'][1] * ws + bs)
        W2 = W2.at[64 * i:64 * i + 64, 32 * i:32 * i + 32].set(p['l2'][0])
        b2 = b2.at[32 * i:32 * i + 32].set(p['l2'][1])
        W3 = W3.at[32 * i:32 * i + 32, i:i + 1].set(p['l3'][0])
        b3 = b3.at[i].set(p['l3'][1][0])
    po = params['outcome']
    w1s, b1s = bnf(po['bn1'])
    w2s, b2s = bnf(po['bn2'])
    W1 = W1.at[:, 256:320].set(po['l1'][0] * w1s[None, :])
    b1 = b1.at[256:320].set(po['l1'][1] * w1s + b1s)
    W2 = W2.at[256:320, 128:160].set(po['l2'][0] * w2s[None, :])
    b2 = b2.at[128:160].set(po['l2'][1] * w2s + b2s)
    W3 = W3.at[128:160, 4:7].set(po['l3'][0])
    b3 = b3.at[4:7].set(po['l3'][1])

    return (W0, b0[None, :], convs,
            W1, b1[None, :], W2, b2[None, :], W3, b3[None, :])


def kernel(x, edge_index, game_indices, params):
    src = edge_index[0].reshape(NC * NS, NCHUNK, C)
    dst = edge_index[1].reshape(NC * NS, NCHUNK, C)
    x = jnp.pad(x, ((0, NP - N), (0, 0)))
    gi = jnp.pad(game_indices.astype(jnp.int32), (0, GP - G))

    (W0, b0, convs, W1c, b1c, W2c, b2c, W3c, b3c) = _fold(params)
    (W1f, b1f), (W2f, b2f), (W3f, b3f) = convs

    onesr = jnp.ones((C, H), jnp.float32)
    zrows = jnp.zeros((C, H), jnp.float32)
    zdeg = jnp.zeros((ZR, H), jnp.float32)

    _sc_degree, _sc_layer, _sc_layer3 = _sc_kernels()
    degp = _sc_degree(dst, onesr, zdeg)
    h0, g1, dinv = _tc_a(x, W0, b0, W1f, degp)

    parts1 = _sc_layer(g1, src, dst, zrows, gi, h0, dinv)
    h1, g2 = _tc_b(h0, g1, parts1, dinv, b1f[None, :], W2f)

    parts2 = _sc_layer(g2, src, dst, zrows, gi, h1, dinv)
    h2, g3 = _tc_b(h1, g2, parts2, dinv, b2f[None, :], W3f)

    aggg, hg, gg, dvg = _sc_layer3(g3, src, dst, zrows, gi, h2, dinv)
    out = _tc_heads(aggg, hg, gg, dvg, b3f[None, :],
                    W1c, b1c, W2c, b2c, W3c, b3c)

    o = out[:G]
    return (o[:, 0:1], o[:, 1:2], o[:, 2:3], o[:, 3:4], o[:, 4:7])
